# baseline (speedup 1.0000x reference)
import functools

import jax
import jax.numpy as jnp
from jax import lax
from jax.experimental import pallas as pl
from jax.experimental.pallas import tpu as pltpu
from jax.experimental.pallas import tpu_sc as plsc

N_NODES = 10000
D = 128
NUM_LAYERS = 3

NC = 2
NS = 16
NW = NC * NS
CHUNK = 128
NB = 2
QTRS = 2
ACC_ROWS = 10240
ROWS_PER_TILE_ZERO = ACC_ROWS // NS
ROWS_PER_TILE_OUT = ACC_ROWS // NS


def _sc_mesh():
    return plsc.VectorSubcoreMesh(core_axis_name="c", subcore_axis_name="s",
                                  num_cores=NC, num_subcores=NS)


def _segsum_body(n_chunks, t_hbm, src_hbm, dst_hbm, out_hbm,
                 src_v, dst_v, rows_v, zb, acc, gsem, ssem, isem):
    c = lax.axis_index("c")
    s = lax.axis_index("s")
    wid = s * NC + c

    for i in range(16):
        for j in range(D // 16):
            zb[i, pl.ds(j * 16, 16)] = jnp.zeros((16,), jnp.float32)

    def zero_body(k, _):
        pltpu.sync_copy(zb, acc.at[pl.ds(s * ROWS_PER_TILE_ZERO + k * 16, 16)])
        return 0

    lax.fori_loop(0, ROWS_PER_TILE_ZERO // 16, zero_body, 0)

    plsc.subcore_barrier()

    qc = n_chunks // QTRS
    for q in range(QTRS):
        pltpu.sync_copy(src_hbm.at[wid, pl.ds(q * qc, qc)], src_v)
        pltpu.sync_copy(dst_hbm.at[wid, pl.ds(q * qc, qc)], dst_v)

        for b in range(NB - 1):
            pltpu.async_copy(t_hbm.at[src_v.at[b]], rows_v[b], gsem.at[b])

        def chunk_body(jb, _):
            for b in range(NB):
                j = jb + b
                bprev = (b - 1) % NB
                pltpu.make_async_copy(
                    t_hbm.at[src_v.at[j]], rows_v[b], gsem.at[b]).wait()
                pltpu.async_copy(rows_v[b], acc.at[dst_v.at[j]],
                                 ssem.at[b], add=True)

                @pl.when(j >= 1)
                def _():
                    pltpu.make_async_copy(
                        rows_v[bprev], acc.at[dst_v.at[j - 1]],
                        ssem.at[bprev]).wait()

                @pl.when(j + NB - 1 < qc)
                def _():
                    pltpu.async_copy(
                        t_hbm.at[src_v.at[j + NB - 1]], rows_v[bprev],
                        gsem.at[bprev])
            return 0

        lax.fori_loop(0, qc // NB, lambda g, u: chunk_body(g * NB, u), 0)

        pltpu.make_async_copy(
            rows_v[NB - 1], acc.at[dst_v.at[qc - 1]],
            ssem.at[NB - 1]).wait()

    plsc.subcore_barrier()

    pltpu.async_copy(
        acc.at[pl.ds(s * ROWS_PER_TILE_OUT, ROWS_PER_TILE_OUT)],
        out_hbm.at[c, pl.ds(s * ROWS_PER_TILE_OUT, ROWS_PER_TILE_OUT)],
        isem,
    ).wait()


def _make_segsum(n_chunks):
    body = functools.partial(_segsum_body, n_chunks)
    return pl.kernel(
        body,
        out_type=jax.ShapeDtypeStruct((NC, ACC_ROWS, D), jnp.float32),
        mesh=_sc_mesh(),
        scratch_types=[
            pltpu.VMEM((n_chunks // QTRS, CHUNK), jnp.int32),
            pltpu.VMEM((n_chunks // QTRS, CHUNK), jnp.int32),
            [pltpu.VMEM((CHUNK, D), jnp.float32) for _ in range(NB)],
            pltpu.VMEM((16, D), jnp.float32),
            pltpu.VMEM_SHARED((ACC_ROWS, D), jnp.float32),
            pltpu.SemaphoreType.DMA((NB,)),
            pltpu.SemaphoreType.DMA((NB,)),
            pltpu.SemaphoreType.DMA,
        ],
        name="sc_segment_sum",
    )


def _counts_body(n_chunks, dst_hbm, out_hbm, dst_v, ones_v, zb, acc, isem):
    c = lax.axis_index("c")
    s = lax.axis_index("s")
    wid = s * NC + c

    for i in range(16):
        for j in range(D // 16):
            zb[i, pl.ds(j * 16, 16)] = jnp.zeros((16,), jnp.float32)
    for i in range(CHUNK):
        for j in range(D // 16):
            ones_v[i, pl.ds(j * 16, 16)] = jnp.ones((16,), jnp.float32)

    def zero_body(k, _):
        pltpu.sync_copy(zb, acc.at[pl.ds(s * ROWS_PER_TILE_ZERO + k * 16, 16)])
        return 0

    lax.fori_loop(0, ROWS_PER_TILE_ZERO // 16, zero_body, 0)

    pltpu.sync_copy(dst_hbm.at[wid], dst_v)

    plsc.subcore_barrier()

    def chunk_body(j, _):
        pltpu.sync_copy(ones_v, acc.at[dst_v.at[j]], add=True)
        return 0

    lax.fori_loop(0, n_chunks, chunk_body, 0)

    plsc.subcore_barrier()

    pltpu.async_copy(
        acc.at[pl.ds(s * ROWS_PER_TILE_OUT, ROWS_PER_TILE_OUT)],
        out_hbm.at[c, pl.ds(s * ROWS_PER_TILE_OUT, ROWS_PER_TILE_OUT)],
        isem,
    ).wait()


def _make_counts(n_chunks):
    body = functools.partial(_counts_body, n_chunks)
    return pl.kernel(
        body,
        out_type=jax.ShapeDtypeStruct((NC, ACC_ROWS, D), jnp.float32),
        mesh=_sc_mesh(),
        scratch_types=[
            pltpu.VMEM((n_chunks, CHUNK), jnp.int32),
            pltpu.VMEM((CHUNK, D), jnp.float32),
            pltpu.VMEM((16, D), jnp.float32),
            pltpu.VMEM_SHARED((ACC_ROWS, D), jnp.float32),
            pltpu.SemaphoreType.DMA,
        ],
        name="sc_degree_counts",
    )


BLK = 400


def _mmT(a, w):
    return lax.dot_general(a, w, (((1,), (1,)), ((), ())),
                           preferred_element_type=jnp.float32)


def _enc_body(x_ref, encW_ref, encb_ref, Wl_ref, bl_ref, Wr_ref,
              t_ref, r_ref):
    h = jnp.maximum(_mmT(x_ref[...], encW_ref[...]) + encb_ref[...], 0.0)
    t_ref[...] = _mmT(h, Wl_ref[...])
    r_ref[...] = _mmT(h, Wr_ref[...]) + bl_ref[...]


def _combine_mid_body(p_ref, r_ref, inv_ref, Wl_ref, bl_ref, Wr_ref,
                      t_ref, rn_ref):
    h = jnp.maximum((p_ref[0] + p_ref[1]) * inv_ref[...] + r_ref[...], 0.0)
    t_ref[...] = _mmT(h, Wl_ref[...])
    rn_ref[...] = _mmT(h, Wr_ref[...]) + bl_ref[...]


def _combine_first_body(p_ref, r_ref, c_ref, Wl_ref, bl_ref, Wr_ref,
                        t_ref, rn_ref, inv_ref):
    cnt = c_ref[0, :, 0:1] + c_ref[1, :, 0:1]
    inv = 1.0 / jnp.maximum(cnt, 1.0)
    inv_ref[...] = inv
    h = jnp.maximum((p_ref[0] + p_ref[1]) * inv + r_ref[...], 0.0)
    t_ref[...] = _mmT(h, Wl_ref[...])
    rn_ref[...] = _mmT(h, Wr_ref[...]) + bl_ref[...]


def _combine_last_body(p_ref, r_ref, inv_ref, decW_ref, decb_ref, out_ref):
    h = jnp.maximum((p_ref[0] + p_ref[1]) * inv_ref[...] + r_ref[...], 0.0)
    out_ref[...] = _mmT(h, decW_ref[...]) + decb_ref[...]


def _row_spec(width):
    return pl.BlockSpec((BLK, width), lambda i: (i, 0))


def _part_spec(width):
    return pl.BlockSpec((NC, BLK, width), lambda i: (0, i, 0))


def _full_spec(shape):
    return pl.BlockSpec(shape, lambda i: tuple(0 for _ in shape))


def _tc_call(body, in_specs, out_specs, out_shapes):
    return pl.pallas_call(
        body,
        grid=(N_NODES // BLK,),
        in_specs=in_specs,
        out_specs=out_specs,
        out_shape=out_shapes,
    )


def kernel(x, edge_index, enc_W, enc_b, conv_Wl, conv_bl, conv_Wr,
           dec_W, dec_b):
    n_edges = edge_index.shape[1]
    src = edge_index[0].astype(jnp.int32)
    dst = edge_index[1].astype(jnp.int32)

    n_chunks = -(-n_edges // (NW * CHUNK))
    n_chunks = -(-n_chunks // (QTRS * NB)) * (QTRS * NB)
    per_w = n_chunks * CHUNK
    total = NW * per_w
    pad = total - n_edges
    src_p = jnp.pad(src, (0, pad)).reshape(NW, n_chunks, CHUNK)
    dst_p = jnp.pad(dst, (0, pad), constant_values=N_NODES)
    dst_p = dst_p.reshape(NW, n_chunks, CHUNK)

    wspec = _full_spec((D, D))
    bspec = _full_spec((1, D))

    enc_call = _tc_call(
        _enc_body,
        [_row_spec(D), wspec, bspec, wspec, bspec, wspec],
        [_row_spec(D), _row_spec(D)],
        [jax.ShapeDtypeStruct((N_NODES, D), jnp.float32)] * 2,
    )
    t, r = enc_call(x, enc_W, enc_b.reshape(1, D), conv_Wl,
                    conv_bl.reshape(1, D), conv_Wr)

    segsum = _make_segsum(n_chunks)
    counts = _make_counts(n_chunks)

    cpart = counts(dst_p)

    first_call = _tc_call(
        _combine_first_body,
        [_part_spec(D), _row_spec(D), _part_spec(D), wspec, bspec, wspec],
        [_row_spec(D), _row_spec(D), _row_spec(1)],
        [jax.ShapeDtypeStruct((N_NODES, D), jnp.float32),
         jax.ShapeDtypeStruct((N_NODES, D), jnp.float32),
         jax.ShapeDtypeStruct((N_NODES, 1), jnp.float32)],
    )
    mid_call = _tc_call(
        _combine_mid_body,
        [_part_spec(D), _row_spec(D), _row_spec(1), wspec, bspec, wspec],
        [_row_spec(D), _row_spec(D)],
        [jax.ShapeDtypeStruct((N_NODES, D), jnp.float32)] * 2,
    )
    last_call = _tc_call(
        _combine_last_body,
        [_part_spec(D), _row_spec(D), _row_spec(1), wspec, bspec],
        [_row_spec(D)],
        [jax.ShapeDtypeStruct((N_NODES, D), jnp.float32)],
    )

    p = segsum(t, src_p, dst_p)
    t, r, inv = first_call(p, r, cpart, conv_Wl, conv_bl.reshape(1, D),
                           conv_Wr)
    p = segsum(t, src_p, dst_p)
    t, r = mid_call(p, r, inv, conv_Wl, conv_bl.reshape(1, D), conv_Wr)
    p = segsum(t, src_p, dst_p)
    out = last_call(p, r, inv, dec_W, dec_b.reshape(1, D))[0]
    return out


# speedup vs baseline: 1.0702x; 1.0702x over previous
import functools

import jax
import jax.numpy as jnp
from jax import lax
from jax.experimental import pallas as pl
from jax.experimental.pallas import tpu as pltpu
from jax.experimental.pallas import tpu_sc as plsc

N_NODES = 10000
D = 128
NUM_LAYERS = 3

NC = 2
NS = 16
NW = NC * NS
CHUNK = 128
NB = 2
QTRS = 2
ACC_ROWS = 10240
ROWS_PER_TILE_ZERO = ACC_ROWS // NS
ROWS_PER_TILE_OUT = ACC_ROWS // NS


def _sc_mesh():
    return plsc.VectorSubcoreMesh(core_axis_name="c", subcore_axis_name="s",
                                  num_cores=NC, num_subcores=NS)


def _segsum_body(n_chunks, t_hbm, src_hbm, dst_hbm, out_hbm,
                 src_v, dst_v, rows_v, zb, acc, gsem, ssem, isem):
    c = lax.axis_index("c")
    s = lax.axis_index("s")
    wid = s * NC + c

    for i in range(16):
        for j in range(D // 16):
            zb[i, pl.ds(j * 16, 16)] = jnp.zeros((16,), jnp.float32)

    def zero_body(k, _):
        pltpu.sync_copy(zb, acc.at[pl.ds(s * ROWS_PER_TILE_ZERO + k * 16, 16)])
        return 0

    lax.fori_loop(0, ROWS_PER_TILE_ZERO // 16, zero_body, 0)

    plsc.subcore_barrier()

    qc = n_chunks // QTRS
    for q in range(QTRS):
        pltpu.sync_copy(src_hbm.at[wid, pl.ds(q * qc, qc)], src_v)
        pltpu.sync_copy(dst_hbm.at[wid, pl.ds(q * qc, qc)], dst_v)

        for b in range(NB):
            pltpu.async_copy(t_hbm.at[src_v.at[b]], rows_v[b], gsem.at[b])

        def chunk_body(jb, _):
            for b in range(NB):
                j = jb + b
                pltpu.make_async_copy(
                    t_hbm.at[src_v.at[j]], rows_v[b], gsem.at[b]).wait()
                pltpu.sync_copy(rows_v[b], acc.at[dst_v.at[j]], add=True)

                @pl.when(j + NB < qc)
                def _():
                    pltpu.async_copy(
                        t_hbm.at[src_v.at[j + NB]], rows_v[b], gsem.at[b])
            return 0

        lax.fori_loop(0, qc // NB, lambda g, u: chunk_body(g * NB, u), 0)

    plsc.subcore_barrier()

    pltpu.async_copy(
        acc.at[pl.ds(s * ROWS_PER_TILE_OUT, ROWS_PER_TILE_OUT)],
        out_hbm.at[c, pl.ds(s * ROWS_PER_TILE_OUT, ROWS_PER_TILE_OUT)],
        isem,
    ).wait()


def _make_segsum(n_chunks):
    body = functools.partial(_segsum_body, n_chunks)
    return pl.kernel(
        body,
        out_type=jax.ShapeDtypeStruct((NC, ACC_ROWS, D), jnp.float32),
        mesh=_sc_mesh(),
        scratch_types=[
            pltpu.VMEM((n_chunks // QTRS, CHUNK), jnp.int32),
            pltpu.VMEM((n_chunks // QTRS, CHUNK), jnp.int32),
            [pltpu.VMEM((CHUNK, D), jnp.float32) for _ in range(NB)],
            pltpu.VMEM((16, D), jnp.float32),
            pltpu.VMEM_SHARED((ACC_ROWS, D), jnp.float32),
            pltpu.SemaphoreType.DMA((NB,)),
            pltpu.SemaphoreType.DMA((NB,)),
            pltpu.SemaphoreType.DMA,
        ],
        name="sc_segment_sum",
    )


def _counts_body(n_chunks, dst_hbm, out_hbm, dst_v, ones_v, zb, acc, isem):
    c = lax.axis_index("c")
    s = lax.axis_index("s")
    wid = s * NC + c

    for i in range(16):
        for j in range(D // 16):
            zb[i, pl.ds(j * 16, 16)] = jnp.zeros((16,), jnp.float32)
    for i in range(CHUNK):
        for j in range(D // 16):
            ones_v[i, pl.ds(j * 16, 16)] = jnp.ones((16,), jnp.float32)

    def zero_body(k, _):
        pltpu.sync_copy(zb, acc.at[pl.ds(s * ROWS_PER_TILE_ZERO + k * 16, 16)])
        return 0

    lax.fori_loop(0, ROWS_PER_TILE_ZERO // 16, zero_body, 0)

    pltpu.sync_copy(dst_hbm.at[wid], dst_v)

    plsc.subcore_barrier()

    def chunk_body(j, _):
        pltpu.sync_copy(ones_v, acc.at[dst_v.at[j]], add=True)
        return 0

    lax.fori_loop(0, n_chunks, chunk_body, 0)

    plsc.subcore_barrier()

    pltpu.async_copy(
        acc.at[pl.ds(s * ROWS_PER_TILE_OUT, ROWS_PER_TILE_OUT)],
        out_hbm.at[c, pl.ds(s * ROWS_PER_TILE_OUT, ROWS_PER_TILE_OUT)],
        isem,
    ).wait()


def _make_counts(n_chunks):
    body = functools.partial(_counts_body, n_chunks)
    return pl.kernel(
        body,
        out_type=jax.ShapeDtypeStruct((NC, ACC_ROWS, D), jnp.float32),
        mesh=_sc_mesh(),
        scratch_types=[
            pltpu.VMEM((n_chunks, CHUNK), jnp.int32),
            pltpu.VMEM((CHUNK, D), jnp.float32),
            pltpu.VMEM((16, D), jnp.float32),
            pltpu.VMEM_SHARED((ACC_ROWS, D), jnp.float32),
            pltpu.SemaphoreType.DMA,
        ],
        name="sc_degree_counts",
    )


BLK = 400


def _mmT(a, w):
    return lax.dot_general(a, w, (((1,), (1,)), ((), ())),
                           preferred_element_type=jnp.float32)


def _enc_body(x_ref, encW_ref, encb_ref, Wl_ref, bl_ref, Wr_ref,
              t_ref, r_ref):
    h = jnp.maximum(_mmT(x_ref[...], encW_ref[...]) + encb_ref[...], 0.0)
    t_ref[...] = _mmT(h, Wl_ref[...])
    r_ref[...] = _mmT(h, Wr_ref[...]) + bl_ref[...]


def _combine_mid_body(p_ref, r_ref, inv_ref, Wl_ref, bl_ref, Wr_ref,
                      t_ref, rn_ref):
    h = jnp.maximum((p_ref[0] + p_ref[1]) * inv_ref[...] + r_ref[...], 0.0)
    t_ref[...] = _mmT(h, Wl_ref[...])
    rn_ref[...] = _mmT(h, Wr_ref[...]) + bl_ref[...]


def _combine_first_body(p_ref, r_ref, c_ref, Wl_ref, bl_ref, Wr_ref,
                        t_ref, rn_ref, inv_ref):
    cnt = c_ref[0, :, 0:1] + c_ref[1, :, 0:1]
    inv = 1.0 / jnp.maximum(cnt, 1.0)
    inv_ref[...] = inv
    h = jnp.maximum((p_ref[0] + p_ref[1]) * inv + r_ref[...], 0.0)
    t_ref[...] = _mmT(h, Wl_ref[...])
    rn_ref[...] = _mmT(h, Wr_ref[...]) + bl_ref[...]


def _combine_last_body(p_ref, r_ref, inv_ref, decW_ref, decb_ref, out_ref):
    h = jnp.maximum((p_ref[0] + p_ref[1]) * inv_ref[...] + r_ref[...], 0.0)
    out_ref[...] = _mmT(h, decW_ref[...]) + decb_ref[...]


def _row_spec(width):
    return pl.BlockSpec((BLK, width), lambda i: (i, 0))


def _part_spec(width):
    return pl.BlockSpec((NC, BLK, width), lambda i: (0, i, 0))


def _full_spec(shape):
    return pl.BlockSpec(shape, lambda i: tuple(0 for _ in shape))


def _tc_call(body, in_specs, out_specs, out_shapes):
    return pl.pallas_call(
        body,
        grid=(N_NODES // BLK,),
        in_specs=in_specs,
        out_specs=out_specs,
        out_shape=out_shapes,
    )


def kernel(x, edge_index, enc_W, enc_b, conv_Wl, conv_bl, conv_Wr,
           dec_W, dec_b):
    n_edges = edge_index.shape[1]
    src = edge_index[0].astype(jnp.int32)
    dst = edge_index[1].astype(jnp.int32)

    n_chunks = -(-n_edges // (NW * CHUNK))
    n_chunks = -(-n_chunks // (QTRS * NB)) * (QTRS * NB)
    per_w = n_chunks * CHUNK
    total = NW * per_w
    pad = total - n_edges
    src_p = jnp.pad(src, (0, pad)).reshape(NW, n_chunks, CHUNK)
    dst_p = jnp.pad(dst, (0, pad), constant_values=N_NODES)
    dst_p = dst_p.reshape(NW, n_chunks, CHUNK)

    wspec = _full_spec((D, D))
    bspec = _full_spec((1, D))

    enc_call = _tc_call(
        _enc_body,
        [_row_spec(D), wspec, bspec, wspec, bspec, wspec],
        [_row_spec(D), _row_spec(D)],
        [jax.ShapeDtypeStruct((N_NODES, D), jnp.float32)] * 2,
    )
    t, r = enc_call(x, enc_W, enc_b.reshape(1, D), conv_Wl,
                    conv_bl.reshape(1, D), conv_Wr)

    segsum = _make_segsum(n_chunks)
    counts = _make_counts(n_chunks)

    cpart = counts(dst_p)

    first_call = _tc_call(
        _combine_first_body,
        [_part_spec(D), _row_spec(D), _part_spec(D), wspec, bspec, wspec],
        [_row_spec(D), _row_spec(D), _row_spec(1)],
        [jax.ShapeDtypeStruct((N_NODES, D), jnp.float32),
         jax.ShapeDtypeStruct((N_NODES, D), jnp.float32),
         jax.ShapeDtypeStruct((N_NODES, 1), jnp.float32)],
    )
    mid_call = _tc_call(
        _combine_mid_body,
        [_part_spec(D), _row_spec(D), _row_spec(1), wspec, bspec, wspec],
        [_row_spec(D), _row_spec(D)],
        [jax.ShapeDtypeStruct((N_NODES, D), jnp.float32)] * 2,
    )
    last_call = _tc_call(
        _combine_last_body,
        [_part_spec(D), _row_spec(D), _row_spec(1), wspec, bspec],
        [_row_spec(D)],
        [jax.ShapeDtypeStruct((N_NODES, D), jnp.float32)],
    )

    p = segsum(t, src_p, dst_p)
    t, r, inv = first_call(p, r, cpart, conv_Wl, conv_bl.reshape(1, D),
                           conv_Wr)
    p = segsum(t, src_p, dst_p)
    t, r = mid_call(p, r, inv, conv_Wl, conv_bl.reshape(1, D), conv_Wr)
    p = segsum(t, src_p, dst_p)
    out = last_call(p, r, inv, dec_W, dec_b.reshape(1, D))[0]
    return out


# speedup vs baseline: 1.1760x; 1.0988x over previous
import functools

import jax
import jax.numpy as jnp
from jax import lax
from jax.experimental import pallas as pl
from jax.experimental.pallas import tpu as pltpu
from jax.experimental.pallas import tpu_sc as plsc

N_NODES = 10000
D = 128
NUM_LAYERS = 3

NC = 2
NS = 16
NW = NC * NS
CHUNK = 128
NB = 2
HALVES = 2
ACC_ROWS = 10240
ROWS_PER_TILE = ACC_ROWS // NS


def _sc_mesh():
    return plsc.VectorSubcoreMesh(core_axis_name="c", subcore_axis_name="s",
                                  num_cores=NC, num_subcores=NS)


def _segsum_body(n_chunks, t_hbm, src_hbm, dst_hbm, out_hbm,
                 src_v, dst_v, rows_v, zb, acc, gsem, isem):
    c = lax.axis_index("c")
    s = lax.axis_index("s")
    wid = s * NC + c

    for i in range(16):
        for j in range(D // 16):
            zb[i, pl.ds(j * 16, 16)] = jnp.zeros((16,), jnp.float32)

    def zero_body(k, _):
        pltpu.sync_copy(zb, acc.at[pl.ds(s * ROWS_PER_TILE + k * 16, 16)])
        return 0

    lax.fori_loop(0, ROWS_PER_TILE // 16, zero_body, 0)

    plsc.subcore_barrier()

    hc = n_chunks // HALVES
    for h in range(HALVES):
        pltpu.sync_copy(src_hbm.at[wid, pl.ds(h * hc, hc)], src_v)
        pltpu.sync_copy(dst_hbm.at[wid, pl.ds(h * hc, hc)], dst_v)

        for b in range(NB):
            pltpu.async_copy(t_hbm.at[src_v.at[b]], rows_v[b], gsem.at[b])

        def chunk_body(jb, _):
            for b in range(NB):
                j = jb + b
                pltpu.make_async_copy(
                    t_hbm.at[src_v.at[j]], rows_v[b], gsem.at[b]).wait()
                pltpu.sync_copy(rows_v[b], acc.at[dst_v.at[j]], add=True)

                @pl.when(j + NB < hc)
                def _():
                    pltpu.async_copy(
                        t_hbm.at[src_v.at[j + NB]], rows_v[b], gsem.at[b])
            return 0

        lax.fori_loop(0, hc // NB, lambda g, u: chunk_body(g * NB, u), 0)

    plsc.subcore_barrier()

    pltpu.async_copy(
        acc.at[pl.ds(s * ROWS_PER_TILE, ROWS_PER_TILE)],
        out_hbm.at[c, pl.ds(s * ROWS_PER_TILE, ROWS_PER_TILE)],
        isem,
    ).wait()


def _make_segsum(n_chunks):
    body = functools.partial(_segsum_body, n_chunks)
    return pl.kernel(
        body,
        out_type=jax.ShapeDtypeStruct((NC, ACC_ROWS, D), jnp.float32),
        mesh=_sc_mesh(),
        scratch_types=[
            pltpu.VMEM((n_chunks // HALVES, CHUNK), jnp.int32),
            pltpu.VMEM((n_chunks // HALVES, CHUNK), jnp.int32),
            [pltpu.VMEM((CHUNK, D), jnp.float32) for _ in range(NB)],
            pltpu.VMEM((16, D), jnp.float32),
            pltpu.VMEM_SHARED((ACC_ROWS, D), jnp.float32),
            pltpu.SemaphoreType.DMA((NB,)),
            pltpu.SemaphoreType.DMA,
        ],
        name="sc_segment_sum",
    )


def _counts_body(n_chunks, dst_hbm, out_hbm, dst_v, ones_v, zb, acc, isem):
    c = lax.axis_index("c")
    s = lax.axis_index("s")
    wid = s * NC + c

    for i in range(16):
        for j in range(D // 16):
            zb[i, pl.ds(j * 16, 16)] = jnp.zeros((16,), jnp.float32)
    for i in range(CHUNK):
        for j in range(D // 16):
            ones_v[i, pl.ds(j * 16, 16)] = jnp.ones((16,), jnp.float32)

    def zero_body(k, _):
        pltpu.sync_copy(zb, acc.at[pl.ds(s * ROWS_PER_TILE + k * 16, 16)])
        return 0

    lax.fori_loop(0, ROWS_PER_TILE // 16, zero_body, 0)

    pltpu.sync_copy(dst_hbm.at[wid], dst_v)

    plsc.subcore_barrier()

    def chunk_body(j, _):
        pltpu.sync_copy(ones_v, acc.at[dst_v.at[j]], add=True)
        return 0

    lax.fori_loop(0, n_chunks, chunk_body, 0)

    plsc.subcore_barrier()

    pltpu.async_copy(
        acc.at[pl.ds(s * ROWS_PER_TILE, ROWS_PER_TILE)],
        out_hbm.at[c, pl.ds(s * ROWS_PER_TILE, ROWS_PER_TILE)],
        isem,
    ).wait()


def _make_counts(n_chunks):
    body = functools.partial(_counts_body, n_chunks)
    return pl.kernel(
        body,
        out_type=jax.ShapeDtypeStruct((NC, ACC_ROWS, D), jnp.float32),
        mesh=_sc_mesh(),
        scratch_types=[
            pltpu.VMEM((n_chunks, CHUNK), jnp.int32),
            pltpu.VMEM((CHUNK, D), jnp.float32),
            pltpu.VMEM((16, D), jnp.float32),
            pltpu.VMEM_SHARED((ACC_ROWS, D), jnp.float32),
            pltpu.SemaphoreType.DMA,
        ],
        name="sc_degree_counts",
    )


BLK = 400


def _mmT(a, w):
    return lax.dot_general(a, w, (((1,), (1,)), ((), ())),
                           preferred_element_type=jnp.float32)


def _enc_body(x_ref, encW_ref, encb_ref, Wl_ref, bl_ref, Wr_ref,
              t_ref, r_ref):
    h = jnp.maximum(_mmT(x_ref[...], encW_ref[...]) + encb_ref[...], 0.0)
    t_ref[...] = _mmT(h, Wl_ref[...])
    r_ref[...] = _mmT(h, Wr_ref[...]) + bl_ref[...]


def _combine_mid_body(p0_ref, p1_ref, r_ref, inv_ref, Wl_ref, bl_ref, Wr_ref,
                      t_ref, rn_ref):
    h = jnp.maximum((p0_ref[...] + p1_ref[...]) * inv_ref[...] + r_ref[...],
                    0.0)
    t_ref[...] = _mmT(h, Wl_ref[...])
    rn_ref[...] = _mmT(h, Wr_ref[...]) + bl_ref[...]


def _combine_first_body(p0_ref, p1_ref, r_ref, c0_ref, c1_ref,
                        Wl_ref, bl_ref, Wr_ref, t_ref, rn_ref, inv_ref):
    cnt = c0_ref[:, 0:1] + c1_ref[:, 0:1]
    inv = 1.0 / jnp.maximum(cnt, 1.0)
    inv_ref[...] = inv
    h = jnp.maximum((p0_ref[...] + p1_ref[...]) * inv + r_ref[...], 0.0)
    t_ref[...] = _mmT(h, Wl_ref[...])
    rn_ref[...] = _mmT(h, Wr_ref[...]) + bl_ref[...]


def _combine_last_body(p0_ref, p1_ref, r_ref, inv_ref, decW_ref, decb_ref,
                       out_ref):
    h = jnp.maximum((p0_ref[...] + p1_ref[...]) * inv_ref[...] + r_ref[...],
                    0.0)
    out_ref[...] = _mmT(h, decW_ref[...]) + decb_ref[...]


def _row_spec(width):
    return pl.BlockSpec((BLK, width), lambda i: (i, 0))


def _full_spec(shape):
    return pl.BlockSpec(shape, lambda i: tuple(0 for _ in shape))


def _tc_call(body, in_specs, out_specs, out_shapes):
    return pl.pallas_call(
        body,
        grid=(N_NODES // BLK,),
        in_specs=in_specs,
        out_specs=out_specs,
        out_shape=out_shapes,
    )


def kernel(x, edge_index, enc_W, enc_b, conv_Wl, conv_bl, conv_Wr,
           dec_W, dec_b):
    n_edges = edge_index.shape[1]
    src = edge_index[0].astype(jnp.int32)
    dst = edge_index[1].astype(jnp.int32)

    n_chunks = -(-n_edges // (NW * CHUNK))
    n_chunks = -(-n_chunks // (HALVES * NB)) * (HALVES * NB)
    per_w = n_chunks * CHUNK
    total = NW * per_w
    pad = total - n_edges
    src_p = jnp.pad(src, (0, pad)).reshape(NW, n_chunks, CHUNK)
    dst_p = jnp.pad(dst, (0, pad), constant_values=N_NODES)
    dst_p = dst_p.reshape(NW, n_chunks, CHUNK)

    wspec = _full_spec((D, D))
    bspec = _full_spec((1, D))

    enc_call = _tc_call(
        _enc_body,
        [_row_spec(D), wspec, bspec, wspec, bspec, wspec],
        [_row_spec(D), _row_spec(D)],
        [jax.ShapeDtypeStruct((N_NODES, D), jnp.float32)] * 2,
    )
    t, r = enc_call(x, enc_W, enc_b.reshape(1, D), conv_Wl,
                    conv_bl.reshape(1, D), conv_Wr)

    segsum = _make_segsum(n_chunks)
    counts = _make_counts(n_chunks)

    cpart = counts(dst_p)[:, :N_NODES, :16]

    first_call = _tc_call(
        _combine_first_body,
        [_row_spec(D), _row_spec(D), _row_spec(D), _row_spec(16),
         _row_spec(16), wspec, bspec, wspec],
        [_row_spec(D), _row_spec(D), _row_spec(1)],
        [jax.ShapeDtypeStruct((N_NODES, D), jnp.float32),
         jax.ShapeDtypeStruct((N_NODES, D), jnp.float32),
         jax.ShapeDtypeStruct((N_NODES, 1), jnp.float32)],
    )
    mid_call = _tc_call(
        _combine_mid_body,
        [_row_spec(D), _row_spec(D), _row_spec(D), _row_spec(1),
         wspec, bspec, wspec],
        [_row_spec(D), _row_spec(D)],
        [jax.ShapeDtypeStruct((N_NODES, D), jnp.float32)] * 2,
    )
    last_call = _tc_call(
        _combine_last_body,
        [_row_spec(D), _row_spec(D), _row_spec(D), _row_spec(1),
         wspec, bspec],
        [_row_spec(D)],
        [jax.ShapeDtypeStruct((N_NODES, D), jnp.float32)],
    )

    p = segsum(t, src_p, dst_p)[:, :N_NODES]
    t, r, inv = first_call(p[0], p[1], r, cpart[0], cpart[1],
                           conv_Wl, conv_bl.reshape(1, D), conv_Wr)
    p = segsum(t, src_p, dst_p)[:, :N_NODES]
    t, r = mid_call(p[0], p[1], r, inv, conv_Wl, conv_bl.reshape(1, D),
                    conv_Wr)
    p = segsum(t, src_p, dst_p)[:, :N_NODES]
    out = last_call(p[0], p[1], r, inv, dec_W, dec_b.reshape(1, D))[0]
    return out
